# baseline (speedup 1.0000x reference)
import functools

import jax
import jax.numpy as jnp
from jax import lax
from jax.experimental import pallas as pl
from jax.experimental.pallas import tpu as pltpu
from jax.experimental.pallas import tpu_sc as plsc

NC = 2
NS = 16
NW = NC * NS

R = 8
NSLOT = 4
LA = 3


def _make_gather(vocab, dim, n):
    assert n % NW == 0
    b_per_w = n // NW
    assert b_per_w % R == 0
    n_groups = b_per_w // R
    assert n_groups >= NSLOT

    mesh = plsc.VectorSubcoreMesh(core_axis_name="c", subcore_axis_name="s")

    @functools.partial(
        pl.kernel,
        out_type=jax.ShapeDtypeStruct((n, dim), jnp.float32),
        mesh=mesh,
        scratch_types=[
            pltpu.VMEM((b_per_w + 16,), jnp.int32),
            pltpu.VMEM_SHARED((NS, NSLOT, R, dim), jnp.float32),
            [pltpu.SemaphoreType.DMA for _ in range(NSLOT)],
            [pltpu.SemaphoreType.DMA for _ in range(NSLOT)],
        ],
    )
    def gather(table_hbm, idx_hbm, out_hbm, idx_v, shared, gsems, ssems):
        cid = lax.axis_index("c")
        sid = lax.axis_index("s")
        wid = sid * NC + cid
        base = wid * b_per_w
        pltpu.sync_copy(idx_hbm.at[pl.ds(base, b_per_w)],
                        idx_v.at[pl.ds(0, b_per_w)])

        def issue_group(g, slot):
            vec = idx_v[pl.ds(g * R, 16)]
            for j in range(R):
                row = vec[j]
                pltpu.async_copy(
                    table_hbm.at[pl.ds(row, 1)],
                    shared.at[sid, slot, pl.ds(j, 1)],
                    gsems[slot],
                )

        for g in range(LA):
            issue_group(g, g % NSLOT)

        def body(g, _):
            nxt = g + LA
            for slot in range(NSLOT):
                @pl.when(lax.rem(nxt, NSLOT) == slot)
                def _():
                    @pl.when(nxt < n_groups)
                    def _():
                        @pl.when(nxt >= NSLOT)
                        def _():
                            pltpu.make_async_copy(
                                shared.at[sid, slot],
                                out_hbm.at[pl.ds(base, R)],
                                ssems[slot],
                            ).wait()
                        issue_group(nxt, slot)
            for slot in range(NSLOT):
                @pl.when(lax.rem(g, NSLOT) == slot)
                def _():
                    pltpu.make_async_copy(
                        table_hbm.at[pl.ds(0, R)], shared.at[sid, slot],
                        gsems[slot],
                    ).wait()
                    pltpu.async_copy(
                        shared.at[sid, slot],
                        out_hbm.at[pl.ds(base + g * R, R)],
                        ssems[slot],
                    )
            return 0

        lax.fori_loop(0, n_groups, body, 0)

        for slot in range(NSLOT):
            pltpu.make_async_copy(
                shared.at[sid, slot], out_hbm.at[pl.ds(base, R)], ssems[slot]
            ).wait()

    return gather


def kernel(input_ids, weight):
    b, s = input_ids.shape
    vocab, dim = weight.shape
    idx = input_ids.reshape(-1).astype(jnp.int32)
    out = _make_gather(vocab, dim, idx.shape[0])(weight, idx)
    return out.reshape(b, s, dim)


# speedup vs baseline: 1.0122x; 1.0122x over previous
import functools

import jax
import jax.numpy as jnp
from jax import lax
from jax.experimental import pallas as pl
from jax.experimental.pallas import tpu as pltpu
from jax.experimental.pallas import tpu_sc as plsc

NC = 2
NS = 16
NW = NC * NS

R = 8
NSLOT = 4
LA = 2


def _make_gather(vocab, dim, n):
    assert n % NW == 0
    b_per_w = n // NW
    assert b_per_w % R == 0
    n_groups = b_per_w // R
    assert n_groups >= NSLOT

    mesh = plsc.VectorSubcoreMesh(core_axis_name="c", subcore_axis_name="s")

    @functools.partial(
        pl.kernel,
        out_type=jax.ShapeDtypeStruct((n, dim), jnp.float32),
        mesh=mesh,
        scratch_types=[
            pltpu.VMEM((b_per_w + 16,), jnp.int32),
            pltpu.VMEM_SHARED((NS, NSLOT, R, dim), jnp.float32),
            [pltpu.SemaphoreType.DMA for _ in range(NSLOT)],
            [pltpu.SemaphoreType.DMA for _ in range(NSLOT)],
        ],
    )
    def gather(table_hbm, idx_hbm, out_hbm, idx_v, shared, gsems, ssems):
        cid = lax.axis_index("c")
        sid = lax.axis_index("s")
        wid = sid * NC + cid
        base = wid * b_per_w
        pltpu.sync_copy(idx_hbm.at[pl.ds(base, b_per_w)],
                        idx_v.at[pl.ds(0, b_per_w)])

        def issue_group(g, slot):
            vec = idx_v[pl.ds(g * R, 16)]
            for j in range(R):
                row = vec[j]
                pltpu.async_copy(
                    table_hbm.at[pl.ds(row, 1)],
                    shared.at[sid, slot, pl.ds(j, 1)],
                    gsems[slot],
                )

        for g in range(LA):
            issue_group(g, g % NSLOT)

        def body(g, _):
            nxt = g + LA
            for slot in range(NSLOT):
                @pl.when(lax.rem(nxt, NSLOT) == slot)
                def _():
                    @pl.when(nxt < n_groups)
                    def _():
                        @pl.when(nxt >= NSLOT)
                        def _():
                            pltpu.make_async_copy(
                                shared.at[sid, slot],
                                out_hbm.at[pl.ds(base, R)],
                                ssems[slot],
                            ).wait()
                        issue_group(nxt, slot)
            for slot in range(NSLOT):
                @pl.when(lax.rem(g, NSLOT) == slot)
                def _():
                    pltpu.make_async_copy(
                        table_hbm.at[pl.ds(0, R)], shared.at[sid, slot],
                        gsems[slot],
                    ).wait()
                    pltpu.async_copy(
                        shared.at[sid, slot],
                        out_hbm.at[pl.ds(base + g * R, R)],
                        ssems[slot],
                    )
            return 0

        lax.fori_loop(0, n_groups, body, 0)

        for slot in range(NSLOT):
            pltpu.make_async_copy(
                shared.at[sid, slot], out_hbm.at[pl.ds(base, R)], ssems[slot]
            ).wait()

    return gather


def kernel(input_ids, weight):
    b, s = input_ids.shape
    vocab, dim = weight.shape
    idx = input_ids.reshape(-1).astype(jnp.int32)
    out = _make_gather(vocab, dim, idx.shape[0])(weight, idx)
    return out.reshape(b, s, dim)
